# baseline (speedup 1.0000x reference)
import functools

import jax
import jax.numpy as jnp
from jax import lax
from jax.experimental import pallas as pl
from jax.experimental.pallas import tpu as pltpu
from jax.experimental.pallas import tpu_sc as plsc

_F = 26
_V = 100000
_D = 64
_B = 16384

_NC = 2
_NS = 16
_NW = _NC * _NS

_LOOKUPS = _B * _F
_PER_W = _LOOKUPS // _NW
_CH = 128
_NCH = _PER_W // _CH
_LANES = 16


def _body(x_hbm, tab_hbm, out_hbm, xv, rows_v, sem):
    wid = lax.axis_index("s") * _NC + lax.axis_index("c")
    base = wid * _PER_W

    pltpu.sync_copy(x_hbm.at[wid], xv)

    lane = lax.broadcasted_iota(jnp.int32, (_LANES,), 0)

    def add_offsets(r, carry):
        for c in range(_CH // _LANES):
            q0 = r * _CH + c * _LANES
            pos = lane + q0
            field = lax.rem(pos, _F)
            sl = pl.ds(c * _LANES, _LANES)
            xv[r, sl] = xv[r, sl] + field * _V
        return carry

    lax.fori_loop(0, _NCH, add_offsets, 0)

    def gather_chunk(j, carry):
        pltpu.async_copy(tab_hbm.at[xv.at[j]], rows_v, sem).wait()
        pltpu.sync_copy(rows_v, out_hbm.at[pl.ds(base + j * _CH, _CH)])
        return carry

    lax.fori_loop(0, _NCH, gather_chunk, 0)


def kernel(x, emb):
    xr = x.astype(jnp.int32).reshape(_NW, _NCH, _CH)
    tab = emb.reshape(_F * _V, _D)
    mesh = plsc.VectorSubcoreMesh(core_axis_name="c", subcore_axis_name="s")
    k = functools.partial(
        pl.kernel,
        mesh=mesh,
        out_type=jax.ShapeDtypeStruct((_LOOKUPS, _D), jnp.float32),
        compiler_params=pltpu.CompilerParams(use_tc_tiling_on_sc=False),
        scratch_types=[
            pltpu.VMEM((_NCH, _CH), jnp.int32),
            pltpu.VMEM((_CH, _D), jnp.float32),
            pltpu.SemaphoreType.DMA,
        ],
    )(_body)
    out = k(xr, tab)
    return out.reshape(_B, _F * _D)


# speedup vs baseline: 1.0425x; 1.0425x over previous
import functools

import jax
import jax.numpy as jnp
from jax import lax
from jax.experimental import pallas as pl
from jax.experimental.pallas import tpu as pltpu
from jax.experimental.pallas import tpu_sc as plsc

_F = 26
_V = 100000
_D = 64
_B = 16384

_NC = 2
_NS = 16
_NW = _NC * _NS

_LOOKUPS = _B * _F
_PER_W = _LOOKUPS // _NW
_CH = 128
_NCH = _PER_W // _CH
_K = 4
_SLAB = _K * _CH
_NSTEP = _PER_W // _SLAB
_NBLK = _LOOKUPS // _CH
_LANES = 16


def _body(x_hbm, tab_hbm, out_hbm, xv, rows0, rows1, sg0, sg1, sw0, sw1):
    wid = lax.axis_index("s") * _NC + lax.axis_index("c")
    blk_base = wid * _NCH

    pltpu.sync_copy(x_hbm.at[wid], xv)

    lane = lax.broadcasted_iota(jnp.int32, (_LANES,), 0)

    def add_offsets(r, carry):
        for c in range(_CH // _LANES):
            q0 = r * _CH + c * _LANES
            pos = lane + q0
            field = lax.rem(pos, _F)
            sl = pl.ds(c * _LANES, _LANES)
            xv[r, sl] = xv[r, sl] + field * _V
        return carry

    lax.fori_loop(0, _NCH, add_offsets, 0)

    def fire(g, rows, sg):
        copies = []
        for kk in range(_K):
            copies.append(
                pltpu.async_copy(tab_hbm.at[xv.at[g * _K + kk]], rows.at[kk], sg)
            )
        return copies

    def start_wb(g, rows, sw):
        return pltpu.async_copy(rows, out_hbm.at[pl.ds(blk_base + g * _K, _K)], sw)

    def wait_wb(rows, sw):
        pltpu.make_async_copy(rows, out_hbm.at[pl.ds(0, _K)], sw).wait()

    def step(g, rows, sg, sw, first):
        if not first:
            wait_wb(rows, sw)
        for c in fire(g, rows, sg):
            c.wait()
        start_wb(g, rows, sw)

    step(0, rows0, sg0, sw0, True)
    step(1, rows1, sg1, sw1, True)

    def pair(m, carry):
        g = 2 + 2 * m
        step(g, rows0, sg0, sw0, False)
        step(g + 1, rows1, sg1, sw1, False)
        return carry

    lax.fori_loop(0, (_NSTEP - 2) // 2, pair, 0)

    wait_wb(rows0, sw0)
    wait_wb(rows1, sw1)


def kernel(x, emb):
    xr = x.astype(jnp.int32).reshape(_NW, _NCH, _CH)
    tab = emb.reshape(_F * _V, _D)
    mesh = plsc.VectorSubcoreMesh(core_axis_name="c", subcore_axis_name="s")
    k = functools.partial(
        pl.kernel,
        mesh=mesh,
        out_type=jax.ShapeDtypeStruct((_NBLK, _CH, _D), jnp.float32),
        compiler_params=pltpu.CompilerParams(use_tc_tiling_on_sc=False),
        scratch_types=[
            pltpu.VMEM((_NCH, _CH), jnp.int32),
            pltpu.VMEM((_K, _CH, _D), jnp.float32),
            pltpu.VMEM((_K, _CH, _D), jnp.float32),
            pltpu.SemaphoreType.DMA,
            pltpu.SemaphoreType.DMA,
            pltpu.SemaphoreType.DMA,
            pltpu.SemaphoreType.DMA,
        ],
    )(_body)
    out = k(xr, tab)
    return out.reshape(_B, _F * _D)


# speedup vs baseline: 1.1709x; 1.1231x over previous
import functools

import jax
import jax.numpy as jnp
from jax import lax
from jax.experimental import pallas as pl
from jax.experimental.pallas import tpu as pltpu
from jax.experimental.pallas import tpu_sc as plsc

_F = 26
_V = 100000
_D = 64
_B = 16384

_NC = 2
_NS = 16
_NW = _NC * _NS

_LOOKUPS = _B * _F
_PER_W = _LOOKUPS // _NW
_CH = 128
_NCH = _PER_W // _CH
_K = 4
_SLAB = _K * _CH
_NSTEP = _PER_W // _SLAB
_NBLK = _LOOKUPS // _CH
_LANES = 16


def _body(x_hbm, tab_hbm, out_hbm, xv, rows0, rows1, sg0, sg1, sw0, sw1):
    wid = lax.axis_index("s") * _NC + lax.axis_index("c")
    blk_base = wid * _NCH

    pltpu.sync_copy(x_hbm.at[wid], xv)

    lane = lax.broadcasted_iota(jnp.int32, (_LANES,), 0)

    def add_offsets(r, carry):
        for c in range(_CH // _LANES):
            q0 = r * _CH + c * _LANES
            pos = lane + q0
            field = lax.rem(pos, _F)
            sl = pl.ds(c * _LANES, _LANES)
            xv[r, sl] = (xv[r, sl] + field * _V) * 2
        return carry

    lax.fori_loop(0, _NCH, add_offsets, 0)

    def fire(g, rows, sg):
        copies = []
        for kk in range(_K):
            copies.append(
                pltpu.async_copy(tab_hbm.at[xv.at[g * _K + kk]], rows.at[kk], sg)
            )
        return copies

    def start_wb(g, rows, sw):
        return pltpu.async_copy(rows, out_hbm.at[pl.ds(blk_base + g * _K, _K)], sw)

    def wait_wb(rows, sw):
        pltpu.make_async_copy(rows, out_hbm.at[pl.ds(0, _K)], sw).wait()

    def step(g, rows, sg, sw, first):
        if not first:
            wait_wb(rows, sw)
        for c in fire(g, rows, sg):
            c.wait()
        start_wb(g, rows, sw)

    step(0, rows0, sg0, sw0, True)
    step(1, rows1, sg1, sw1, True)

    def pair(m, carry):
        g = 2 + 2 * m
        step(g, rows0, sg0, sw0, False)
        step(g + 1, rows1, sg1, sw1, False)
        return carry

    lax.fori_loop(0, (_NSTEP - 2) // 2, pair, 0)

    wait_wb(rows0, sw0)
    wait_wb(rows1, sw1)


def kernel(x, emb):
    xr = x.astype(jnp.int32).reshape(_NW, _NCH, _CH)
    epad = jnp.pad(emb, ((0, 0), (0, 0), (0, _D)))
    tab = epad.reshape(_F * _V * 2, _D)
    mesh = plsc.VectorSubcoreMesh(core_axis_name="c", subcore_axis_name="s")
    k = functools.partial(
        pl.kernel,
        mesh=mesh,
        out_type=jax.ShapeDtypeStruct((_NBLK, _CH, _D), jnp.float32),
        compiler_params=pltpu.CompilerParams(use_tc_tiling_on_sc=False),
        scratch_types=[
            pltpu.VMEM((_NCH, _CH), jnp.int32),
            pltpu.VMEM((_K, _CH, _D), jnp.float32),
            pltpu.VMEM((_K, _CH, _D), jnp.float32),
            pltpu.SemaphoreType.DMA,
            pltpu.SemaphoreType.DMA,
            pltpu.SemaphoreType.DMA,
            pltpu.SemaphoreType.DMA,
        ],
    )(_body)
    out = k(xr, tab)
    return out.reshape(_B, _F * _D)
